# Initial kernel scaffold; baseline (speedup 1.0000x reference)
import jax
import jax.numpy as jnp
from jax.experimental import pallas as pl


def kernel(cls_prob_0, cls_prob_1, cls_prob_2, cls_prob_3, cls_prob_4, bbox_pred_0, bbox_pred_1, bbox_pred_2, bbox_pred_3, bbox_pred_4, im_info):
    raise NotImplementedError("write your pallas kernel here")


# speedup vs baseline: 18.1176x; 18.1176x over previous
import functools

import numpy as np
import jax
import jax.numpy as jnp
from jax import lax
from jax.experimental import pallas as pl
from jax.experimental.pallas import tpu as pltpu

_FEAT_STRIDES = (4, 8, 16, 32, 64)
_RATIOS = (0.5, 1.0, 2.0)
_SCALES = (8.0,)
_A = 3
_PRE = 2000
_POST = 1000
_NMS_THRESH = 0.7
_MIN_SIZE = 4.0
_SIZES = (256, 128, 64, 32, 16)
_N = sum(_A * s * s for s in _SIZES)
_LANES = 128
_ROWS = _N // _LANES
_NPAD = 2048


def _anchor_geometry():
    ws_all, hs_all, cx_all, cy_all = [], [], [], []
    for stride, size in zip(_FEAT_STRIDES, _SIZES):
        base = np.array([0.0, 0.0, stride - 1.0, stride - 1.0])
        w = base[2] - base[0] + 1.0
        h = base[3] - base[1] + 1.0
        xc = base[0] + 0.5 * (w - 1.0)
        yc = base[1] + 0.5 * (h - 1.0)
        sz = w * h
        anchors = []
        for r in _RATIOS:
            aw = np.round(np.sqrt(sz / r))
            ah = np.round(aw * r)
            for s in _SCALES:
                aws = aw * s
                ahs = ah * s
                anchors.append([xc - 0.5 * (aws - 1.0), yc - 0.5 * (ahs - 1.0),
                                xc + 0.5 * (aws - 1.0), yc + 0.5 * (ahs - 1.0)])
        anchors_base = np.array(anchors, dtype=np.float32)
        sx = np.arange(size, dtype=np.float32) * stride
        sy = np.arange(size, dtype=np.float32) * stride
        gx, gy = np.meshgrid(sx, sy)
        shifts = np.stack([gx.ravel(), gy.ravel(), gx.ravel(), gy.ravel()], axis=1)
        anc = (shifts[:, None, :] + anchors_base[None, :, :]).reshape(-1, 4)
        aw = anc[:, 2] - anc[:, 0] + 1.0
        ah = anc[:, 3] - anc[:, 1] + 1.0
        ws_all.append(aw)
        hs_all.append(ah)
        cx_all.append(anc[:, 0] + 0.5 * (aw - 1.0))
        cy_all.append(anc[:, 1] + 0.5 * (ah - 1.0))
    cat = lambda xs: np.concatenate(xs).astype(np.float32).reshape(_ROWS, _LANES)
    return cat(ws_all), cat(hs_all), cat(cx_all), cat(cy_all)


_AW, _AH, _ACX, _ACY = _anchor_geometry()


def _decode_body(im_ref, sc_ref, dx_ref, dy_ref, dw_ref, dh_ref,
                 aw_ref, ah_ref, cx_ref, cy_ref,
                 x1_ref, y1_ref, x2_ref, y2_ref, so_ref):
    im_h = im_ref[0, 0]
    im_w = im_ref[0, 1]
    aw = aw_ref[...]
    ah = ah_ref[...]
    pcx = dx_ref[...] * aw + cx_ref[...]
    pcy = dy_ref[...] * ah + cy_ref[...]
    pw = jnp.exp(jnp.minimum(dw_ref[...], 10.0)) * aw
    ph = jnp.exp(jnp.minimum(dh_ref[...], 10.0)) * ah
    x1 = jnp.clip(pcx - 0.5 * (pw - 1.0), 0.0, im_w - 1.0)
    y1 = jnp.clip(pcy - 0.5 * (ph - 1.0), 0.0, im_h - 1.0)
    x2 = jnp.clip(pcx + 0.5 * (pw - 1.0), 0.0, im_w - 1.0)
    y2 = jnp.clip(pcy + 0.5 * (ph - 1.0), 0.0, im_h - 1.0)
    ok = ((x2 - x1 + 1.0) >= _MIN_SIZE) & ((y2 - y1 + 1.0) >= _MIN_SIZE)
    x1_ref[...] = x1
    y1_ref[...] = y1
    x2_ref[...] = x2
    y2_ref[...] = y2
    so_ref[...] = jnp.where(ok, sc_ref[...], -1e9)


_f32 = lambda shape: jax.ShapeDtypeStruct(shape, jnp.float32)

_decode = pl.pallas_call(
    _decode_body,
    in_specs=[pl.BlockSpec(memory_space=pltpu.SMEM)] +
             [pl.BlockSpec(memory_space=pltpu.VMEM)] * 9,
    out_specs=[pl.BlockSpec(memory_space=pltpu.VMEM)] * 5,
    out_shape=[_f32((_ROWS, _LANES))] * 5,
)


def _nms_body(x1s, y1s, x2s, y2s, x1v, y1v, x2v, y2v, keep_ref):
    x1 = x1v[...]
    y1 = y1v[...]
    x2 = x2v[...]
    y2 = y2v[...]
    areas = (x2 - x1 + 1.0) * (y2 - y1 + 1.0)
    row = lax.broadcasted_iota(jnp.int32, (_NPAD // _LANES, _LANES), 0)
    col = lax.broadcasted_iota(jnp.int32, (_NPAD // _LANES, _LANES), 1)
    idx = row * _LANES + col
    keep0 = jnp.where(idx < _PRE, 1.0, 0.0)

    def body(i, keep):
        bx1 = x1s[i]
        by1 = y1s[i]
        bx2 = x2s[i]
        by2 = y2s[i]
        area_i = (bx2 - bx1 + 1.0) * (by2 - by1 + 1.0)
        xx1 = jnp.maximum(bx1, x1)
        yy1 = jnp.maximum(by1, y1)
        xx2 = jnp.minimum(bx2, x2)
        yy2 = jnp.minimum(by2, y2)
        w = jnp.maximum(0.0, xx2 - xx1 + 1.0)
        h = jnp.maximum(0.0, yy2 - yy1 + 1.0)
        inter = w * h
        iou = inter / (area_i + areas - inter)
        keep_i = jnp.sum(jnp.where(idx == i, keep, 0.0))
        sup = (iou > _NMS_THRESH) & (idx > i) & (keep_i > 0.0)
        return jnp.where(sup, 0.0, keep)

    keep_ref[...] = lax.fori_loop(0, _PRE, body, keep0, unroll=False)


_nms = pl.pallas_call(
    _nms_body,
    in_specs=[pl.BlockSpec(memory_space=pltpu.SMEM)] * 4 +
             [pl.BlockSpec(memory_space=pltpu.VMEM)] * 4,
    out_specs=pl.BlockSpec(memory_space=pltpu.VMEM),
    out_shape=_f32((_NPAD // _LANES, _LANES)),
)


def kernel(cls_prob_0, cls_prob_1, cls_prob_2, cls_prob_3, cls_prob_4,
           bbox_pred_0, bbox_pred_1, bbox_pred_2, bbox_pred_3, bbox_pred_4,
           im_info):
    scs, dls = [], []
    for cls, bbox in zip(
            (cls_prob_0, cls_prob_1, cls_prob_2, cls_prob_3, cls_prob_4),
            (bbox_pred_0, bbox_pred_1, bbox_pred_2, bbox_pred_3, bbox_pred_4)):
        scs.append(jnp.transpose(cls[0, _A:, :, :], (1, 2, 0)).reshape(-1))
        dls.append(jnp.transpose(bbox[0], (1, 2, 0)).reshape(-1, 4))
    scores_in = jnp.concatenate(scs).reshape(_ROWS, _LANES)
    deltas = jnp.concatenate(dls, axis=0)
    dx = deltas[:, 0].reshape(_ROWS, _LANES)
    dy = deltas[:, 1].reshape(_ROWS, _LANES)
    dw = deltas[:, 2].reshape(_ROWS, _LANES)
    dh = deltas[:, 3].reshape(_ROWS, _LANES)

    x1, y1, x2, y2, scores = _decode(
        im_info, scores_in, dx, dy, dw, dh,
        jnp.asarray(_AW), jnp.asarray(_AH), jnp.asarray(_ACX), jnp.asarray(_ACY))

    top_scores, top_idx = lax.top_k(scores.reshape(-1), _PRE)
    bx1 = x1.reshape(-1)[top_idx]
    by1 = y1.reshape(-1)[top_idx]
    bx2 = x2.reshape(-1)[top_idx]
    by2 = y2.reshape(-1)[top_idx]

    pad = _NPAD - _PRE
    px1 = jnp.pad(bx1, (0, pad))
    py1 = jnp.pad(by1, (0, pad))
    px2 = jnp.pad(bx2, (0, pad))
    py2 = jnp.pad(by2, (0, pad))
    shape2d = (_NPAD // _LANES, _LANES)

    keep = _nms(px1, py1, px2, py2,
                px1.reshape(shape2d), py1.reshape(shape2d),
                px2.reshape(shape2d), py2.reshape(shape2d)).reshape(-1)[:_PRE]

    final_scores = jnp.where(keep > 0.0, top_scores, -1e9)
    post_scores, post_idx = lax.top_k(final_scores, _POST)
    rois = jnp.stack([jnp.zeros((_POST,), jnp.float32),
                      bx1[post_idx], by1[post_idx], bx2[post_idx], by2[post_idx]],
                     axis=1)
    return rois, post_scores


# speedup vs baseline: 23.9024x; 1.3193x over previous
import functools

import numpy as np
import jax
import jax.numpy as jnp
from jax import lax
from jax.experimental import pallas as pl
from jax.experimental.pallas import tpu as pltpu

_FEAT_STRIDES = (4, 8, 16, 32, 64)
_RATIOS = (0.5, 1.0, 2.0)
_SCALES = (8.0,)
_A = 3
_PRE = 2000
_POST = 1000
_NMS_THRESH = 0.7
_MIN_SIZE = 4.0
_SIZES = (256, 128, 64, 32, 16)
_N = sum(_A * s * s for s in _SIZES)
_LANES = 128
_ROWS = _N // _LANES
_NPAD = 2048

_RPP = tuple(s * s // _LANES for s in _SIZES)
_LEVEL_ROW_OFF = tuple(int(x) for x in np.cumsum([0] + [_A * r for r in _RPP]))
_LEVEL_ELEM_OFF = tuple(o * _LANES for o in _LEVEL_ROW_OFF)


def _anchor_geometry():
    ws_all, hs_all, cx_all, cy_all = [], [], [], []
    for stride, size in zip(_FEAT_STRIDES, _SIZES):
        base = np.array([0.0, 0.0, stride - 1.0, stride - 1.0])
        w = base[2] - base[0] + 1.0
        h = base[3] - base[1] + 1.0
        xc = base[0] + 0.5 * (w - 1.0)
        yc = base[1] + 0.5 * (h - 1.0)
        sz = w * h
        anchors = []
        for r in _RATIOS:
            aw = np.round(np.sqrt(sz / r))
            ah = np.round(aw * r)
            for s in _SCALES:
                aws = aw * s
                ahs = ah * s
                anchors.append([xc - 0.5 * (aws - 1.0), yc - 0.5 * (ahs - 1.0),
                                xc + 0.5 * (aws - 1.0), yc + 0.5 * (ahs - 1.0)])
        anchors_base = np.array(anchors, dtype=np.float32)
        sx = np.arange(size, dtype=np.float32) * stride
        sy = np.arange(size, dtype=np.float32) * stride
        gx, gy = np.meshgrid(sx, sy)
        shifts = np.stack([gx.ravel(), gy.ravel(), gx.ravel(), gy.ravel()], axis=1)
        anc = (anchors_base[:, None, :] + shifts[None, :, :]).reshape(-1, 4)
        aw = anc[:, 2] - anc[:, 0] + 1.0
        ah = anc[:, 3] - anc[:, 1] + 1.0
        ws_all.append(aw)
        hs_all.append(ah)
        cx_all.append(anc[:, 0] + 0.5 * (aw - 1.0))
        cy_all.append(anc[:, 1] + 0.5 * (ah - 1.0))
    cat = lambda xs: np.concatenate(xs).astype(np.float32).reshape(_ROWS, _LANES)
    return cat(ws_all), cat(hs_all), cat(cx_all), cat(cy_all)


_AW, _AH, _ACX, _ACY = _anchor_geometry()


def _decode_body(im_ref, c0, c1, c2, c3, c4, b0, b1, b2, b3, b4,
                 aw_ref, ah_ref, cx_ref, cy_ref,
                 x1_ref, y1_ref, x2_ref, y2_ref, so_ref):
    im_h = im_ref[0, 0]
    im_w = im_ref[0, 1]
    for lvl, (cls, bbox) in enumerate(zip((c0, c1, c2, c3, c4),
                                          (b0, b1, b2, b3, b4))):
        rpp = _RPP[lvl]
        for a in range(_A):
            out_r = _LEVEL_ROW_OFF[lvl] + a * rpp
            sl_out = slice(out_r, out_r + rpp)
            sc = cls[(_A + a) * rpp:(_A + a + 1) * rpp, :]
            dx = bbox[(4 * a + 0) * rpp:(4 * a + 1) * rpp, :]
            dy = bbox[(4 * a + 1) * rpp:(4 * a + 2) * rpp, :]
            dw = bbox[(4 * a + 2) * rpp:(4 * a + 3) * rpp, :]
            dh = bbox[(4 * a + 3) * rpp:(4 * a + 4) * rpp, :]
            aw = aw_ref[sl_out, :]
            ah = ah_ref[sl_out, :]
            pcx = dx * aw + cx_ref[sl_out, :]
            pcy = dy * ah + cy_ref[sl_out, :]
            pw = jnp.exp(jnp.minimum(dw, 10.0)) * aw
            ph = jnp.exp(jnp.minimum(dh, 10.0)) * ah
            x1 = jnp.clip(pcx - 0.5 * (pw - 1.0), 0.0, im_w - 1.0)
            y1 = jnp.clip(pcy - 0.5 * (ph - 1.0), 0.0, im_h - 1.0)
            x2 = jnp.clip(pcx + 0.5 * (pw - 1.0), 0.0, im_w - 1.0)
            y2 = jnp.clip(pcy + 0.5 * (ph - 1.0), 0.0, im_h - 1.0)
            ok = ((x2 - x1 + 1.0) >= _MIN_SIZE) & ((y2 - y1 + 1.0) >= _MIN_SIZE)
            x1_ref[sl_out, :] = x1
            y1_ref[sl_out, :] = y1
            x2_ref[sl_out, :] = x2
            y2_ref[sl_out, :] = y2
            so_ref[sl_out, :] = jnp.where(ok, sc, -1e9)


_f32 = lambda shape: jax.ShapeDtypeStruct(shape, jnp.float32)

_decode = pl.pallas_call(
    _decode_body,
    in_specs=[pl.BlockSpec(memory_space=pltpu.SMEM)] +
             [pl.BlockSpec(memory_space=pltpu.VMEM)] * 14,
    out_specs=[pl.BlockSpec(memory_space=pltpu.VMEM)] * 5,
    out_shape=[_f32((_ROWS, _LANES))] * 5,
)


def _nms_body(x1s, y1s, x2s, y2s, x1v, y1v, x2v, y2v, keep_ref):
    x1 = x1v[...]
    y1 = y1v[...]
    x2 = x2v[...]
    y2 = y2v[...]
    areas = (x2 - x1 + 1.0) * (y2 - y1 + 1.0)
    row = lax.broadcasted_iota(jnp.int32, (_NPAD // _LANES, _LANES), 0)
    col = lax.broadcasted_iota(jnp.int32, (_NPAD // _LANES, _LANES), 1)
    idx = row * _LANES + col
    keep0 = jnp.where(idx < _PRE, 1.0, 0.0)

    def body(i, keep):
        bx1 = x1s[i]
        by1 = y1s[i]
        bx2 = x2s[i]
        by2 = y2s[i]
        area_i = (bx2 - bx1 + 1.0) * (by2 - by1 + 1.0)
        xx1 = jnp.maximum(bx1, x1)
        yy1 = jnp.maximum(by1, y1)
        xx2 = jnp.minimum(bx2, x2)
        yy2 = jnp.minimum(by2, y2)
        w = jnp.maximum(0.0, xx2 - xx1 + 1.0)
        h = jnp.maximum(0.0, yy2 - yy1 + 1.0)
        inter = w * h
        iou = inter / (area_i + areas - inter)
        keep_i = jnp.sum(jnp.where(idx == i, keep, 0.0))
        sup = (iou > _NMS_THRESH) & (idx > i) & (keep_i > 0.0)
        return jnp.where(sup, 0.0, keep)

    keep_ref[...] = lax.fori_loop(0, _PRE, body, keep0, unroll=False)


_nms = pl.pallas_call(
    _nms_body,
    in_specs=[pl.BlockSpec(memory_space=pltpu.SMEM)] * 4 +
             [pl.BlockSpec(memory_space=pltpu.VMEM)] * 4,
    out_specs=pl.BlockSpec(memory_space=pltpu.VMEM),
    out_shape=_f32((_NPAD // _LANES, _LANES)),
)


def kernel(cls_prob_0, cls_prob_1, cls_prob_2, cls_prob_3, cls_prob_4,
           bbox_pred_0, bbox_pred_1, bbox_pred_2, bbox_pred_3, bbox_pred_4,
           im_info):
    clss = [c.reshape(6 * r, _LANES) for c, r in
            zip((cls_prob_0, cls_prob_1, cls_prob_2, cls_prob_3, cls_prob_4), _RPP)]
    bbxs = [b.reshape(12 * r, _LANES) for b, r in
            zip((bbox_pred_0, bbox_pred_1, bbox_pred_2, bbox_pred_3, bbox_pred_4), _RPP)]

    x1, y1, x2, y2, scores = _decode(
        im_info, *clss, *bbxs,
        jnp.asarray(_AW), jnp.asarray(_AH), jnp.asarray(_ACX), jnp.asarray(_ACY))

    ref_order = []
    for lvl in range(5):
        r0 = _LEVEL_ROW_OFF[lvl]
        blk = scores[r0:r0 + _A * _RPP[lvl], :].reshape(_A, -1)
        ref_order.append(jnp.transpose(blk).reshape(-1))
    scores_ref_order = jnp.concatenate(ref_order)
    top_scores, top_idx = lax.top_k(scores_ref_order, _PRE)

    lvl_of = jnp.searchsorted(
        jnp.asarray(_LEVEL_ELEM_OFF[1:], dtype=jnp.int32), top_idx, side="right")
    e_off = jnp.asarray(_LEVEL_ELEM_OFF, dtype=jnp.int32)[lvl_of]
    plane = jnp.asarray([r * _LANES for r in _RPP], dtype=jnp.int32)[lvl_of]
    r = top_idx - e_off
    sidx = e_off + (r % _A) * plane + r // _A

    bx1 = x1.reshape(-1)[sidx]
    by1 = y1.reshape(-1)[sidx]
    bx2 = x2.reshape(-1)[sidx]
    by2 = y2.reshape(-1)[sidx]

    pad = _NPAD - _PRE
    px1 = jnp.pad(bx1, (0, pad))
    py1 = jnp.pad(by1, (0, pad))
    px2 = jnp.pad(bx2, (0, pad))
    py2 = jnp.pad(by2, (0, pad))
    shape2d = (_NPAD // _LANES, _LANES)

    keep = _nms(px1, py1, px2, py2,
                px1.reshape(shape2d), py1.reshape(shape2d),
                px2.reshape(shape2d), py2.reshape(shape2d)).reshape(-1)[:_PRE]

    final_scores = jnp.where(keep > 0.0, top_scores, -1e9)
    post_scores, post_idx = lax.top_k(final_scores, _POST)
    rois = jnp.stack([jnp.zeros((_POST,), jnp.float32),
                      bx1[post_idx], by1[post_idx], bx2[post_idx], by2[post_idx]],
                     axis=1)
    return rois, post_scores


# speedup vs baseline: 24.2425x; 1.0142x over previous
import functools

import numpy as np
import jax
import jax.numpy as jnp
from jax import lax
from jax.experimental import pallas as pl
from jax.experimental.pallas import tpu as pltpu

_FEAT_STRIDES = (4, 8, 16, 32, 64)
_RATIOS = (0.5, 1.0, 2.0)
_SCALES = (8.0,)
_A = 3
_PRE = 2000
_POST = 1000
_NMS_THRESH = 0.7
_MIN_SIZE = 4.0
_SIZES = (256, 128, 64, 32, 16)
_N = sum(_A * s * s for s in _SIZES)
_LANES = 128
_ROWS = _N // _LANES
_NPAD = 2048

_RPP = tuple(s * s // _LANES for s in _SIZES)
_LEVEL_ROW_OFF = tuple(int(x) for x in np.cumsum([0] + [_A * r for r in _RPP]))
_LEVEL_ELEM_OFF = tuple(o * _LANES for o in _LEVEL_ROW_OFF)


def _anchor_geometry():
    ws_all, hs_all, cx_all, cy_all = [], [], [], []
    for stride, size in zip(_FEAT_STRIDES, _SIZES):
        base = np.array([0.0, 0.0, stride - 1.0, stride - 1.0])
        w = base[2] - base[0] + 1.0
        h = base[3] - base[1] + 1.0
        xc = base[0] + 0.5 * (w - 1.0)
        yc = base[1] + 0.5 * (h - 1.0)
        sz = w * h
        anchors = []
        for r in _RATIOS:
            aw = np.round(np.sqrt(sz / r))
            ah = np.round(aw * r)
            for s in _SCALES:
                aws = aw * s
                ahs = ah * s
                anchors.append([xc - 0.5 * (aws - 1.0), yc - 0.5 * (ahs - 1.0),
                                xc + 0.5 * (aws - 1.0), yc + 0.5 * (ahs - 1.0)])
        anchors_base = np.array(anchors, dtype=np.float32)
        sx = np.arange(size, dtype=np.float32) * stride
        sy = np.arange(size, dtype=np.float32) * stride
        gx, gy = np.meshgrid(sx, sy)
        shifts = np.stack([gx.ravel(), gy.ravel(), gx.ravel(), gy.ravel()], axis=1)
        anc = (anchors_base[:, None, :] + shifts[None, :, :]).reshape(-1, 4)
        aw = anc[:, 2] - anc[:, 0] + 1.0
        ah = anc[:, 3] - anc[:, 1] + 1.0
        ws_all.append(aw)
        hs_all.append(ah)
        cx_all.append(anc[:, 0] + 0.5 * (aw - 1.0))
        cy_all.append(anc[:, 1] + 0.5 * (ah - 1.0))
    cat = lambda xs: np.concatenate(xs).astype(np.float32).reshape(_ROWS, _LANES)
    return cat(ws_all), cat(hs_all), cat(cx_all), cat(cy_all)


_AW, _AH, _ACX, _ACY = _anchor_geometry()


def _decode_body(im_ref, c0, c1, c2, c3, c4, b0, b1, b2, b3, b4,
                 aw_ref, ah_ref, cx_ref, cy_ref,
                 x1_ref, y1_ref, x2_ref, y2_ref, so_ref):
    im_h = im_ref[0, 0]
    im_w = im_ref[0, 1]
    for lvl, (cls, bbox) in enumerate(zip((c0, c1, c2, c3, c4),
                                          (b0, b1, b2, b3, b4))):
        rpp = _RPP[lvl]
        for a in range(_A):
            out_r = _LEVEL_ROW_OFF[lvl] + a * rpp
            sl_out = slice(out_r, out_r + rpp)
            sc = cls[(_A + a) * rpp:(_A + a + 1) * rpp, :]
            dx = bbox[(4 * a + 0) * rpp:(4 * a + 1) * rpp, :]
            dy = bbox[(4 * a + 1) * rpp:(4 * a + 2) * rpp, :]
            dw = bbox[(4 * a + 2) * rpp:(4 * a + 3) * rpp, :]
            dh = bbox[(4 * a + 3) * rpp:(4 * a + 4) * rpp, :]
            aw = aw_ref[sl_out, :]
            ah = ah_ref[sl_out, :]
            pcx = dx * aw + cx_ref[sl_out, :]
            pcy = dy * ah + cy_ref[sl_out, :]
            pw = jnp.exp(jnp.minimum(dw, 10.0)) * aw
            ph = jnp.exp(jnp.minimum(dh, 10.0)) * ah
            x1 = jnp.clip(pcx - 0.5 * (pw - 1.0), 0.0, im_w - 1.0)
            y1 = jnp.clip(pcy - 0.5 * (ph - 1.0), 0.0, im_h - 1.0)
            x2 = jnp.clip(pcx + 0.5 * (pw - 1.0), 0.0, im_w - 1.0)
            y2 = jnp.clip(pcy + 0.5 * (ph - 1.0), 0.0, im_h - 1.0)
            ok = ((x2 - x1 + 1.0) >= _MIN_SIZE) & ((y2 - y1 + 1.0) >= _MIN_SIZE)
            x1_ref[sl_out, :] = x1
            y1_ref[sl_out, :] = y1
            x2_ref[sl_out, :] = x2
            y2_ref[sl_out, :] = y2
            so_ref[sl_out, :] = jnp.where(ok, sc, -1e9)


_f32 = lambda shape: jax.ShapeDtypeStruct(shape, jnp.float32)

_decode = pl.pallas_call(
    _decode_body,
    in_specs=[pl.BlockSpec(memory_space=pltpu.SMEM)] +
             [pl.BlockSpec(memory_space=pltpu.VMEM)] * 14,
    out_specs=[pl.BlockSpec(memory_space=pltpu.VMEM)] * 5,
    out_shape=[_f32((_ROWS, _LANES))] * 5,
)


def _nms_body(x1s, y1s, x2s, y2s, x1v, y1v, x2v, y2v, keep_ref):
    x1 = x1v[...]
    y1 = y1v[...]
    x2 = x2v[...]
    y2 = y2v[...]
    areas = (x2 - x1 + 1.0) * (y2 - y1 + 1.0)
    row = lax.broadcasted_iota(jnp.int32, (_NPAD // _LANES, _LANES), 0)
    col = lax.broadcasted_iota(jnp.int32, (_NPAD // _LANES, _LANES), 1)
    idx = row * _LANES + col
    keep0 = jnp.where(idx < _PRE, 1.0, 0.0)

    def body(i, keep):
        bx1 = x1s[i]
        by1 = y1s[i]
        bx2 = x2s[i]
        by2 = y2s[i]
        area_i = (bx2 - bx1 + 1.0) * (by2 - by1 + 1.0)
        xx1 = jnp.maximum(bx1, x1)
        yy1 = jnp.maximum(by1, y1)
        xx2 = jnp.minimum(bx2, x2)
        yy2 = jnp.minimum(by2, y2)
        w = jnp.maximum(0.0, xx2 - xx1 + 1.0)
        h = jnp.maximum(0.0, yy2 - yy1 + 1.0)
        inter = w * h
        iou = inter / (area_i + areas - inter)
        keep_i = jnp.sum(jnp.where(idx == i, keep, 0.0))
        sup = (iou > _NMS_THRESH) & (idx > i) & (keep_i > 0.0)
        return jnp.where(sup, 0.0, keep)

    keep_ref[...] = lax.fori_loop(0, _PRE, body, keep0, unroll=4)


_nms = pl.pallas_call(
    _nms_body,
    in_specs=[pl.BlockSpec(memory_space=pltpu.SMEM)] * 4 +
             [pl.BlockSpec(memory_space=pltpu.VMEM)] * 4,
    out_specs=pl.BlockSpec(memory_space=pltpu.VMEM),
    out_shape=_f32((_NPAD // _LANES, _LANES)),
)


def kernel(cls_prob_0, cls_prob_1, cls_prob_2, cls_prob_3, cls_prob_4,
           bbox_pred_0, bbox_pred_1, bbox_pred_2, bbox_pred_3, bbox_pred_4,
           im_info):
    clss = [c.reshape(6 * r, _LANES) for c, r in
            zip((cls_prob_0, cls_prob_1, cls_prob_2, cls_prob_3, cls_prob_4), _RPP)]
    bbxs = [b.reshape(12 * r, _LANES) for b, r in
            zip((bbox_pred_0, bbox_pred_1, bbox_pred_2, bbox_pred_3, bbox_pred_4), _RPP)]

    x1, y1, x2, y2, scores = _decode(
        im_info, *clss, *bbxs,
        jnp.asarray(_AW), jnp.asarray(_AH), jnp.asarray(_ACX), jnp.asarray(_ACY))

    ref_order = []
    for lvl in range(5):
        r0 = _LEVEL_ROW_OFF[lvl]
        blk = scores[r0:r0 + _A * _RPP[lvl], :].reshape(_A, -1)
        ref_order.append(jnp.transpose(blk).reshape(-1))
    scores_ref_order = jnp.concatenate(ref_order)
    top_scores, top_idx = lax.top_k(scores_ref_order, _PRE)

    lvl_of = jnp.searchsorted(
        jnp.asarray(_LEVEL_ELEM_OFF[1:], dtype=jnp.int32), top_idx, side="right")
    e_off = jnp.asarray(_LEVEL_ELEM_OFF, dtype=jnp.int32)[lvl_of]
    plane = jnp.asarray([r * _LANES for r in _RPP], dtype=jnp.int32)[lvl_of]
    r = top_idx - e_off
    sidx = e_off + (r % _A) * plane + r // _A

    bx1 = x1.reshape(-1)[sidx]
    by1 = y1.reshape(-1)[sidx]
    bx2 = x2.reshape(-1)[sidx]
    by2 = y2.reshape(-1)[sidx]

    pad = _NPAD - _PRE
    px1 = jnp.pad(bx1, (0, pad))
    py1 = jnp.pad(by1, (0, pad))
    px2 = jnp.pad(bx2, (0, pad))
    py2 = jnp.pad(by2, (0, pad))
    shape2d = (_NPAD // _LANES, _LANES)

    keep = _nms(px1, py1, px2, py2,
                px1.reshape(shape2d), py1.reshape(shape2d),
                px2.reshape(shape2d), py2.reshape(shape2d)).reshape(-1)[:_PRE]

    final_scores = jnp.where(keep > 0.0, top_scores, -1e9)
    post_scores, post_idx = lax.top_k(final_scores, _POST)
    rois = jnp.stack([jnp.zeros((_POST,), jnp.float32),
                      bx1[post_idx], by1[post_idx], bx2[post_idx], by2[post_idx]],
                     axis=1)
    return rois, post_scores


# speedup vs baseline: 33.9278x; 1.3995x over previous
import functools

import numpy as np
import jax
import jax.numpy as jnp
from jax import lax
from jax.experimental import pallas as pl
from jax.experimental.pallas import tpu as pltpu

_FEAT_STRIDES = (4, 8, 16, 32, 64)
_RATIOS = (0.5, 1.0, 2.0)
_SCALES = (8.0,)
_A = 3
_PRE = 2000
_POST = 1000
_NMS_THRESH = 0.7
_MIN_SIZE = 4.0
_SIZES = (256, 128, 64, 32, 16)
_N = sum(_A * s * s for s in _SIZES)
_LANES = 128
_ROWS = _N // _LANES
_NPAD = 2048

_RPP = tuple(s * s // _LANES for s in _SIZES)
_LEVEL_ROW_OFF = tuple(int(x) for x in np.cumsum([0] + [_A * r for r in _RPP]))
_LEVEL_ELEM_OFF = tuple(o * _LANES for o in _LEVEL_ROW_OFF)


def _anchor_geometry():
    ws_all, hs_all, cx_all, cy_all = [], [], [], []
    for stride, size in zip(_FEAT_STRIDES, _SIZES):
        base = np.array([0.0, 0.0, stride - 1.0, stride - 1.0])
        w = base[2] - base[0] + 1.0
        h = base[3] - base[1] + 1.0
        xc = base[0] + 0.5 * (w - 1.0)
        yc = base[1] + 0.5 * (h - 1.0)
        sz = w * h
        anchors = []
        for r in _RATIOS:
            aw = np.round(np.sqrt(sz / r))
            ah = np.round(aw * r)
            for s in _SCALES:
                aws = aw * s
                ahs = ah * s
                anchors.append([xc - 0.5 * (aws - 1.0), yc - 0.5 * (ahs - 1.0),
                                xc + 0.5 * (aws - 1.0), yc + 0.5 * (ahs - 1.0)])
        anchors_base = np.array(anchors, dtype=np.float32)
        sx = np.arange(size, dtype=np.float32) * stride
        sy = np.arange(size, dtype=np.float32) * stride
        gx, gy = np.meshgrid(sx, sy)
        shifts = np.stack([gx.ravel(), gy.ravel(), gx.ravel(), gy.ravel()], axis=1)
        anc = (anchors_base[:, None, :] + shifts[None, :, :]).reshape(-1, 4)
        aw = anc[:, 2] - anc[:, 0] + 1.0
        ah = anc[:, 3] - anc[:, 1] + 1.0
        ws_all.append(aw)
        hs_all.append(ah)
        cx_all.append(anc[:, 0] + 0.5 * (aw - 1.0))
        cy_all.append(anc[:, 1] + 0.5 * (ah - 1.0))
    cat = lambda xs: np.concatenate(xs).astype(np.float32).reshape(_ROWS, _LANES)
    return cat(ws_all), cat(hs_all), cat(cx_all), cat(cy_all)


_AW, _AH, _ACX, _ACY = _anchor_geometry()


def _decode_body(im_ref, c0, c1, c2, c3, c4, b0, b1, b2, b3, b4,
                 aw_ref, ah_ref, cx_ref, cy_ref,
                 x1_ref, y1_ref, x2_ref, y2_ref, so_ref):
    im_h = im_ref[0, 0]
    im_w = im_ref[0, 1]
    for lvl, (cls, bbox) in enumerate(zip((c0, c1, c2, c3, c4),
                                          (b0, b1, b2, b3, b4))):
        rpp = _RPP[lvl]
        for a in range(_A):
            out_r = _LEVEL_ROW_OFF[lvl] + a * rpp
            sl_out = slice(out_r, out_r + rpp)
            sc = cls[(_A + a) * rpp:(_A + a + 1) * rpp, :]
            dx = bbox[(4 * a + 0) * rpp:(4 * a + 1) * rpp, :]
            dy = bbox[(4 * a + 1) * rpp:(4 * a + 2) * rpp, :]
            dw = bbox[(4 * a + 2) * rpp:(4 * a + 3) * rpp, :]
            dh = bbox[(4 * a + 3) * rpp:(4 * a + 4) * rpp, :]
            aw = aw_ref[sl_out, :]
            ah = ah_ref[sl_out, :]
            pcx = dx * aw + cx_ref[sl_out, :]
            pcy = dy * ah + cy_ref[sl_out, :]
            pw = jnp.exp(jnp.minimum(dw, 10.0)) * aw
            ph = jnp.exp(jnp.minimum(dh, 10.0)) * ah
            x1 = jnp.clip(pcx - 0.5 * (pw - 1.0), 0.0, im_w - 1.0)
            y1 = jnp.clip(pcy - 0.5 * (ph - 1.0), 0.0, im_h - 1.0)
            x2 = jnp.clip(pcx + 0.5 * (pw - 1.0), 0.0, im_w - 1.0)
            y2 = jnp.clip(pcy + 0.5 * (ph - 1.0), 0.0, im_h - 1.0)
            ok = ((x2 - x1 + 1.0) >= _MIN_SIZE) & ((y2 - y1 + 1.0) >= _MIN_SIZE)
            x1_ref[sl_out, :] = x1
            y1_ref[sl_out, :] = y1
            x2_ref[sl_out, :] = x2
            y2_ref[sl_out, :] = y2
            so_ref[sl_out, :] = jnp.where(ok, sc, -1e9)


_f32 = lambda shape: jax.ShapeDtypeStruct(shape, jnp.float32)

_decode = pl.pallas_call(
    _decode_body,
    in_specs=[pl.BlockSpec(memory_space=pltpu.SMEM)] +
             [pl.BlockSpec(memory_space=pltpu.VMEM)] * 14,
    out_specs=[pl.BlockSpec(memory_space=pltpu.VMEM)] * 5,
    out_shape=[_f32((_ROWS, _LANES))] * 5,
)


_NT = _NPAD // _LANES


def _nms_body(x1s, y1s, x2s, y2s, x1l, y1l, x2l, y2l, keep_ref):
    X1 = x1s[...]
    Y1 = y1s[...]
    X2 = x2s[...]
    Y2 = y2s[...]
    a_sub = (X2 - X1 + 1.0) * (Y2 - Y1 + 1.0)
    jrow = lax.broadcasted_iota(jnp.int32, (_NPAD, 1), 0)
    keep = jnp.where(jrow < _PRE, 1.0, 0.0)

    for t in range(_NT):
        p_x1 = x1l[t:t + 1, :]
        p_y1 = y1l[t:t + 1, :]
        p_x2 = x2l[t:t + 1, :]
        p_y2 = y2l[t:t + 1, :]
        p_area = (p_x2 - p_x1 + 1.0) * (p_y2 - p_y1 + 1.0)
        gidx = t * _LANES + lax.broadcasted_iota(jnp.int32, (1, _LANES), 1)
        xx1 = jnp.maximum(X1, p_x1)
        yy1 = jnp.maximum(Y1, p_y1)
        xx2 = jnp.minimum(X2, p_x2)
        yy2 = jnp.minimum(Y2, p_y2)
        w = jnp.maximum(0.0, xx2 - xx1 + 1.0)
        h = jnp.maximum(0.0, yy2 - yy1 + 1.0)
        inter = w * h
        iou = inter / (p_area + a_sub - inter)
        S = jnp.where((iou > _NMS_THRESH) & (jrow > gidx), 1.0, 0.0)
        M = S[t * _LANES:(t + 1) * _LANES, :]
        init_p = keep[t * _LANES:(t + 1) * _LANES, :]

        def fix_body(carry):
            kp, _ = carry
            supp = jnp.dot(M, kp, preferred_element_type=jnp.float32)
            kn = jnp.where(supp > 0.5, 0.0, init_p)
            return kn, jnp.any(kn != kp)

        kp, _ = lax.while_loop(lambda c: c[1], fix_body,
                               (init_p, jnp.bool_(True)))
        sup_all = jnp.dot(S, kp, preferred_element_type=jnp.float32)
        keep = jnp.where(sup_all > 0.5, 0.0, keep)

    keep_ref[...] = keep


_nms = pl.pallas_call(
    _nms_body,
    in_specs=[pl.BlockSpec(memory_space=pltpu.VMEM)] * 8,
    out_specs=pl.BlockSpec(memory_space=pltpu.VMEM),
    out_shape=_f32((_NPAD, 1)),
)


def kernel(cls_prob_0, cls_prob_1, cls_prob_2, cls_prob_3, cls_prob_4,
           bbox_pred_0, bbox_pred_1, bbox_pred_2, bbox_pred_3, bbox_pred_4,
           im_info):
    clss = [c.reshape(6 * r, _LANES) for c, r in
            zip((cls_prob_0, cls_prob_1, cls_prob_2, cls_prob_3, cls_prob_4), _RPP)]
    bbxs = [b.reshape(12 * r, _LANES) for b, r in
            zip((bbox_pred_0, bbox_pred_1, bbox_pred_2, bbox_pred_3, bbox_pred_4), _RPP)]

    x1, y1, x2, y2, scores = _decode(
        im_info, *clss, *bbxs,
        jnp.asarray(_AW), jnp.asarray(_AH), jnp.asarray(_ACX), jnp.asarray(_ACY))

    ref_order = []
    for lvl in range(5):
        r0 = _LEVEL_ROW_OFF[lvl]
        blk = scores[r0:r0 + _A * _RPP[lvl], :].reshape(_A, -1)
        ref_order.append(jnp.transpose(blk).reshape(-1))
    scores_ref_order = jnp.concatenate(ref_order)
    top_scores, top_idx = lax.top_k(scores_ref_order, _PRE)

    lvl_of = jnp.searchsorted(
        jnp.asarray(_LEVEL_ELEM_OFF[1:], dtype=jnp.int32), top_idx, side="right")
    e_off = jnp.asarray(_LEVEL_ELEM_OFF, dtype=jnp.int32)[lvl_of]
    plane = jnp.asarray([r * _LANES for r in _RPP], dtype=jnp.int32)[lvl_of]
    r = top_idx - e_off
    sidx = e_off + (r % _A) * plane + r // _A

    bx1 = x1.reshape(-1)[sidx]
    by1 = y1.reshape(-1)[sidx]
    bx2 = x2.reshape(-1)[sidx]
    by2 = y2.reshape(-1)[sidx]

    pad = _NPAD - _PRE
    px1 = jnp.pad(bx1, (0, pad))
    py1 = jnp.pad(by1, (0, pad))
    px2 = jnp.pad(bx2, (0, pad))
    py2 = jnp.pad(by2, (0, pad))
    shape2d = (_NPAD // _LANES, _LANES)

    keep = _nms(px1.reshape(_NPAD, 1), py1.reshape(_NPAD, 1),
                px2.reshape(_NPAD, 1), py2.reshape(_NPAD, 1),
                px1.reshape(shape2d), py1.reshape(shape2d),
                px2.reshape(shape2d), py2.reshape(shape2d)).reshape(-1)[:_PRE]

    final_scores = jnp.where(keep > 0.0, top_scores, -1e9)
    post_scores, post_idx = lax.top_k(final_scores, _POST)
    rois = jnp.stack([jnp.zeros((_POST,), jnp.float32),
                      bx1[post_idx], by1[post_idx], bx2[post_idx], by2[post_idx]],
                     axis=1)
    return rois, post_scores
